# baseline (speedup 1.0000x reference)
import jax
import jax.numpy as jnp
from jax.experimental import pallas as pl
from jax.experimental.pallas import tpu as pltpu

REP = 64
CHUNK = 8


def _body(ew_ref, pos_ref, out_ref, scratch, sem0, sem1):
    base = ew_ref[...] + pos_ref[...]
    b = out_ref.shape[0]
    sems = (sem0, sem1)
    copies = []
    for c in range(REP // CHUNK):
        for r in range(c * CHUNK, (c + 1) * CHUNK):
            scratch[r] = base
        copies.append(
            pltpu.make_async_copy(
                scratch.at[pl.ds(c * CHUNK, CHUNK)],
                out_ref.at[pl.ds(c * CHUNK, CHUNK)],
                sems[c % 2],
            )
        )
        copies[-1].start()
    for i in range(1, b // REP):
        copies.append(
            pltpu.make_async_copy(
                scratch, out_ref.at[pl.ds(i * REP, REP)], sems[i % 2]
            )
        )
        copies[-1].start()
    for c in copies:
        c.wait()


def kernel(x, embed_weight, pos):
    b = x.shape[0]
    n, d = embed_weight.shape
    return pl.pallas_call(
        _body,
        in_specs=[
            pl.BlockSpec(memory_space=pltpu.VMEM),
            pl.BlockSpec(memory_space=pltpu.VMEM),
        ],
        out_specs=pl.BlockSpec(memory_space=pl.ANY),
        out_shape=jax.ShapeDtypeStruct((b, n, d), jnp.float32),
        scratch_shapes=[
            pltpu.VMEM((REP, n, d), jnp.float32),
            pltpu.SemaphoreType.DMA,
            pltpu.SemaphoreType.DMA,
        ],
    )(embed_weight, pos)
